# baseline (speedup 1.0000x reference)
import functools

import jax
import jax.numpy as jnp
import numpy as np
from jax import lax
from jax.experimental import pallas as pl
from jax.experimental.pallas import tpu as pltpu
from jax.experimental.pallas import tpu_sc as plsc

N_CH = 100
RESOLUTIONS = (16, 64, 256)
DIM = 16
BATCH = 16384
NRES = 3
TBASE = (0, N_CH * (RESOLUTIONS[0] + 1),
         N_CH * (RESOLUTIONS[0] + 1) + N_CH * (RESOLUTIONS[1] + 1))

NUM_CORES = 2
NUM_SUBCORES = 16
NW = NUM_CORES * NUM_SUBCORES
B_PER_W = BATCH // NW
CB = 8
PAIRS = CB * N_CH
GROWS = NRES * PAIRS
N_CHUNKS = B_PER_W // CB
LANES = 16
N_VEC = PAIRS // LANES


def _make_kernel():
    mesh = plsc.VectorSubcoreMesh(core_axis_name="c", subcore_axis_name="s")

    @functools.partial(
        pl.kernel,
        out_type=jax.ShapeDtypeStruct((BATCH * N_CH, NRES, DIM), jnp.float32),
        mesh=mesh,
        compiler_params=pltpu.CompilerParams(use_tc_tiling_on_sc=False),
        scratch_types=[
            pltpu.VMEM((PAIRS,), jnp.float32),
            pltpu.VMEM((PAIRS,), jnp.int32),
            pltpu.VMEM((GROWS,), jnp.int32),
            pltpu.VMEM((GROWS,), jnp.int32),
            pltpu.VMEM((2, GROWS, DIM), jnp.float32),
            pltpu.SemaphoreType.DMA,
            pltpu.SemaphoreType.DMA,
            pltpu.SemaphoreType.DMA,
        ],
    )
    def mre_kernel(x_hbm, tcat_hbm, out_hbm,
                   x_v, ch_v, idx0_v, idx1_v, rows_v, g0, g1, osem):
        idx_v = (idx0_v, idx1_v)
        gsem = (g0, g1)
        wid = lax.axis_index("s") * NUM_CORES + lax.axis_index("c")
        pair0 = wid * (B_PER_W * N_CH)

        def ch_body(i, _):
            v = lax.iota(jnp.int32, LANES) + i * LANES
            ch_v[pl.ds(i * LANES, LANES)] = lax.rem(v, N_CH)
            return 0

        lax.fori_loop(0, N_VEC, ch_body, 0)

        def compute_idx(chunk, b):
            p0 = pair0 + chunk * PAIRS
            pltpu.sync_copy(x_hbm.at[pl.ds(p0, PAIRS)], x_v)
            idx_b = idx_v[b]

            def idx_body(i, _):
                s = pl.ds(i * LANES, LANES)
                xv = x_v[s]
                ch = ch_v[s]
                for ri, res in enumerate(RESOLUTIONS):
                    scale = np.float32(res - 1)
                    step = np.float32(1.0 / (res - 1))
                    t = xv * scale
                    c0 = t.astype(jnp.int32)
                    base = c0 - 1
                    acc = jnp.maximum(base, 0)
                    for j in range(4):
                        k = base + j
                        bk = k.astype(jnp.float32) * step
                        valid = (k >= 0) & (k <= res - 1)
                        acc = acc + jnp.where(valid & (bk < xv),
                                              jnp.int32(1), jnp.int32(0))
                    full = acc + ch * (res + 1) + TBASE[ri]
                    idx_b[pl.ds(ri * PAIRS + i * LANES, LANES)] = full
                return 0

            lax.fori_loop(0, N_VEC, idx_body, 0)

        def issue_gather(b):
            pltpu.async_copy(tcat_hbm.at[idx_v[b]], rows_v.at[b], gsem[b])

        def wait_gather_issue_outs(chunk, b):
            pltpu.make_async_copy(tcat_hbm.at[idx_v[b]], rows_v.at[b],
                                  gsem[b]).wait()
            p0 = pair0 + chunk * PAIRS
            for ri in range(NRES):
                pltpu.async_copy(
                    rows_v.at[b, pl.ds(ri * PAIRS, PAIRS)],
                    out_hbm.at[pl.ds(p0, PAIRS), ri], osem)

        def drain_outs(chunk):
            p0 = pair0 + chunk * PAIRS
            for ri in range(NRES):
                pltpu.make_async_copy(
                    rows_v.at[0, pl.ds(ri * PAIRS, PAIRS)],
                    out_hbm.at[pl.ds(p0, PAIRS), ri], osem).wait()

        def loop_body(chunk, _):
            buf = lax.rem(chunk, 2)

            @pl.when(chunk >= 2)
            def _():
                drain_outs(chunk - 2)

            for b in range(2):
                @pl.when(buf == b)
                def _(b=b):
                    compute_idx(chunk, b)
                    issue_gather(b)

                    @pl.when(chunk >= 1)
                    def _():
                        wait_gather_issue_outs(chunk - 1, 1 - b)
            return 0

        lax.fori_loop(0, N_CHUNKS, loop_body, 0)
        last = N_CHUNKS - 1
        wait_gather_issue_outs(last, last % 2)
        drain_outs(last - 1)
        drain_outs(last)

    return mre_kernel


_MRE = _make_kernel()


def kernel(x, W0, W1, W2):
    tcat = jnp.concatenate([W0, W1, W2], axis=0)
    out = _MRE(x.reshape(-1), tcat)
    return out.reshape(BATCH, N_CH, NRES * DIM)


# speedup vs baseline: 6.8203x; 6.8203x over previous
import functools

import jax
import jax.numpy as jnp
import numpy as np
from jax import lax
from jax.experimental import pallas as pl
from jax.experimental.pallas import tpu as pltpu
from jax.experimental.pallas import tpu_sc as plsc

N_CH = 100
RESOLUTIONS = (16, 64, 256)
DIM = 16
BATCH = 16384
NRES = 3
NDIM = NRES * DIM
TBASE = (0, N_CH * (RESOLUTIONS[0] + 1),
         N_CH * (RESOLUTIONS[0] + 1) + N_CH * (RESOLUTIONS[1] + 1))

NUM_CORES = 2
NUM_SUBCORES = 16
NW = NUM_CORES * NUM_SUBCORES
B_PER_W = BATCH // NW
BB = 128
N_BB = B_PER_W // BB
CC = 4
N_CCHUNK = N_CH // CC
GROWS = CC * NRES * BB
LANES = 16
N_CHUNKS = N_BB * N_CCHUNK


def _make_kernel():
    mesh = plsc.VectorSubcoreMesh(core_axis_name="c", subcore_axis_name="s")

    @functools.partial(
        pl.kernel,
        out_type=jax.ShapeDtypeStruct((N_CH, NDIM, BATCH), jnp.float32),
        mesh=mesh,
        compiler_params=pltpu.CompilerParams(
            use_tc_tiling_on_sc=False, needs_layout_passes=False),
        scratch_types=[
            pltpu.VMEM((N_CH, BB), jnp.float32),
            pltpu.VMEM((GROWS,), jnp.int32),
            pltpu.VMEM((GROWS,), jnp.int32),
            pltpu.VMEM((2, GROWS, DIM), jnp.float32),
            pltpu.VMEM((2, NDIM, BB), jnp.float32),
            pltpu.SemaphoreType.DMA,
            pltpu.SemaphoreType.DMA,
            pltpu.SemaphoreType.DMA,
            pltpu.SemaphoreType.DMA,
        ],
    )
    def mre_kernel(xt_hbm, tcat_hbm, out_hbm,
                   x_v, idx0_v, idx1_v, rows_v, t_v, g0, g1, o0, o1):
        osem = (o0, o1)
        idx_v = (idx0_v, idx1_v)
        gsem = (g0, g1)
        wid = lax.axis_index("s") * NUM_CORES + lax.axis_index("c")
        b_w = wid * B_PER_W
        iot = lax.iota(jnp.int32, LANES)

        def load_x(bb):
            pltpu.sync_copy(xt_hbm.at[:, pl.ds(b_w + bb * BB, BB)], x_v)

        def compute_idx(chunk, b):
            c0 = lax.rem(chunk, N_CCHUNK) * CC
            idx_b = idx_v[b]
            for ri, res in enumerate(RESOLUTIONS):
                scale = np.float32(res - 1)
                step = np.float32(1.0 / (res - 1))

                def idx_body(i, _, ri=ri, res=res, scale=scale, step=step):
                    cc = i // (BB // LANES)
                    g = lax.rem(i, BB // LANES)
                    ch = c0 + cc
                    xv = x_v[ch, pl.ds(g * LANES, LANES)]
                    t = xv * scale
                    c0i = t.astype(jnp.int32)
                    base = c0i - 1
                    acc = jnp.maximum(base, 0)
                    for j in range(4):
                        k = base + j
                        bk = k.astype(jnp.float32) * step
                        valid = (k >= 0) & (k <= res - 1)
                        acc = acc + jnp.where(valid & (bk < xv),
                                              jnp.int32(1), jnp.int32(0))
                    off = ch * (res + 1) + TBASE[ri]
                    pos = (cc * NRES + ri) * BB + g * LANES
                    idx_b[pl.ds(pos, LANES)] = acc + off
                    return 0

                lax.fori_loop(0, CC * (BB // LANES), idx_body, 0)

        def issue_gather(b):
            pltpu.async_copy(tcat_hbm.at[idx_v[b]], rows_v.at[b], gsem[b])

        def transpose_writeout(chunk, b):
            pltpu.make_async_copy(tcat_hbm.at[idx_v[b]], rows_v.at[b],
                                  gsem[b]).wait()
            bb = chunk // N_CCHUNK
            c0 = lax.rem(chunk, N_CCHUNK) * CC
            b0 = b_w + bb * BB
            for cc in range(CC):
                tb = (cc % 2)

                @pl.when(chunk * CC + cc >= 2)
                def _(tb=tb, cc=cc):
                    pltpu.make_async_copy(
                        t_v.at[tb],
                        out_hbm.at[c0 + cc, :, pl.ds(b0, BB)], osem[tb]).wait()

                def t_body(d, _, cc=cc, tb=tb):
                    ri = d // DIM
                    dd = lax.rem(d, DIM)
                    row0 = (cc * NRES + ri) * BB
                    for g in range(BB // LANES):
                        v = plsc.load_gather(
                            rows_v, [jnp.full((LANES,), b, jnp.int32),
                                     row0 + g * LANES + iot,
                                     jnp.full((LANES,), dd, jnp.int32)])
                        t_v[tb, d, pl.ds(g * LANES, LANES)] = v
                    return 0

                lax.fori_loop(0, NDIM, t_body, 0)
                pltpu.async_copy(
                    t_v.at[tb], out_hbm.at[c0 + cc, :, pl.ds(b0, BB)], osem[tb])

        def pipe_body(chunk, _):
            buf = lax.rem(chunk, 2)
            for b in range(2):
                @pl.when(buf == b)
                def _(b=b):
                    @pl.when(lax.rem(chunk, N_CCHUNK) == 0)
                    def _():
                        load_x(chunk // N_CCHUNK)

                    compute_idx(chunk, b)
                    issue_gather(b)

                    @pl.when(chunk >= 1)
                    def _():
                        transpose_writeout(chunk - 1, 1 - b)
            return 0

        lax.fori_loop(0, N_CHUNKS, pipe_body, 0)
        last = N_CHUNKS - 1
        transpose_writeout(last, last % 2)
        bb = last // N_CCHUNK
        c0 = (last % N_CCHUNK) * CC
        b0 = b_w + bb * BB
        for tb in range(2):
            pltpu.make_async_copy(
                t_v.at[tb], out_hbm.at[c0 + 2 + tb, :, pl.ds(b0, BB)],
                osem[tb]).wait()

    return mre_kernel


_MRE = _make_kernel()


def kernel(x, W0, W1, W2):
    tcat = jnp.concatenate([W0, W1, W2], axis=0)
    out = _MRE(x.T, tcat)
    return out.transpose(2, 0, 1)


# speedup vs baseline: 7.4538x; 1.0929x over previous
import functools

import jax
import jax.numpy as jnp
import numpy as np
from jax import lax
from jax.experimental import pallas as pl
from jax.experimental.pallas import tpu as pltpu
from jax.experimental.pallas import tpu_sc as plsc

N_CH = 100
RESOLUTIONS = (16, 64, 256)
DIM = 16
BATCH = 16384
NRES = 3
NDIM = NRES * DIM

NUM_CORES = 2
NUM_SUBCORES = 16
NW = NUM_CORES * NUM_SUBCORES
B_PER_W = BATCH // NW
BB = 128
N_BB = B_PER_W // BB
CC = 4
N_CCHUNK = N_CH // CC
LANES = 16
G_PER_BB = BB // LANES

TFLAT = (0, N_CH * (RESOLUTIONS[0] + 1) * DIM,
         N_CH * ((RESOLUTIONS[0] + 1) + (RESOLUTIONS[1] + 1)) * DIM)
SLICE_LEN = tuple(CC * (r + 1) * DIM for r in RESOLUTIONS)
LBASE = (0, SLICE_LEN[0], SLICE_LEN[0] + SLICE_LEN[1])
WTAB_LEN = sum(SLICE_LEN)
IDX_LEN = CC * NRES * BB


def _make_kernel():
    mesh = plsc.VectorSubcoreMesh(core_axis_name="c", subcore_axis_name="s")

    @functools.partial(
        pl.kernel,
        out_type=jax.ShapeDtypeStruct((N_CH, NDIM, BATCH), jnp.float32),
        mesh=mesh,
        compiler_params=pltpu.CompilerParams(
            use_tc_tiling_on_sc=True, needs_layout_passes=False),
        scratch_types=[
            pltpu.VMEM((N_CH, B_PER_W), jnp.float32),
            pltpu.VMEM((WTAB_LEN,), jnp.float32),
            pltpu.VMEM((IDX_LEN,), jnp.int32),
            pltpu.VMEM((2, NDIM, BB), jnp.float32),
            pltpu.SemaphoreType.DMA,
            pltpu.SemaphoreType.DMA,
        ],
    )
    def mre_kernel(xt_hbm, tflat_hbm, out_hbm, x_v, w_v, idx_v, t_v, o0, o1):
        osem = (o0, o1)
        wid = lax.axis_index("s") * NUM_CORES + lax.axis_index("c")
        b_w = wid * B_PER_W
        iot = lax.iota(jnp.int32, LANES)

        pltpu.sync_copy(xt_hbm.at[:, pl.ds(b_w, B_PER_W)], x_v)

        def load_tables(c0):
            for ri, res in enumerate(RESOLUTIONS):
                rl = (res + 1) * DIM
                pltpu.sync_copy(
                    tflat_hbm.at[pl.ds(TFLAT[ri] + c0 * rl, CC * rl)],
                    w_v.at[pl.ds(LBASE[ri], CC * rl)])

        def compute_idx(c0, bb):
            def idx_body(i, _):
                cc = i // G_PER_BB
                g = lax.rem(i, G_PER_BB)
                xv = x_v[c0 + cc, pl.ds(bb * BB + g * LANES, LANES)]
                for ri, res in enumerate(RESOLUTIONS):
                    scale = np.float32(res - 1)
                    step = np.float32(1.0 / (res - 1))
                    rl = (res + 1) * DIM
                    t = xv * scale
                    c0i = t.astype(jnp.int32)
                    base = c0i - 1
                    acc = jnp.maximum(base, 0)
                    for j in range(4):
                        k = base + j
                        bk = k.astype(jnp.float32) * step
                        valid = (k >= 0) & (k <= res - 1)
                        acc = acc + jnp.where(valid & (bk < xv),
                                              jnp.int32(1), jnp.int32(0))
                    flat = acc * DIM + (LBASE[ri] + cc * rl)
                    pos = (cc * NRES + ri) * BB + g * LANES
                    idx_v[pl.ds(pos, LANES)] = flat
                return 0

            lax.fori_loop(0, CC * G_PER_BB, idx_body, 0)

        def gather_transpose_out(c0, bb, use):
            b0 = b_w + bb * BB
            for cc in range(CC):
                tb = cc % 2

                @pl.when(use + cc >= 2)
                def _(tb=tb, cc=cc):
                    pltpu.make_async_copy(
                        t_v.at[tb],
                        out_hbm.at[c0 + cc, :, pl.ds(b0, BB)], osem[tb]).wait()

                def t_body(i, _, cc=cc, tb=tb):
                    ri = i // G_PER_BB
                    g = lax.rem(i, G_PER_BB)
                    ridx = idx_v[pl.ds((cc * NRES + ri) * BB + g * LANES,
                                       LANES)]
                    for dd in range(DIM):
                        v = plsc.load_gather(w_v, [ridx + dd])
                        t_v[tb, ri * DIM + dd, pl.ds(g * LANES, LANES)] = v
                    return 0

                lax.fori_loop(0, NRES * G_PER_BB, t_body, 0)
                pltpu.async_copy(
                    t_v.at[tb], out_hbm.at[c0 + cc, :, pl.ds(b0, BB)],
                    osem[tb])

        def chunk_body(ci, _):
            c0 = ci * CC
            load_tables(c0)

            def bb_body(bb, _):
                compute_idx(c0, bb)
                use = (ci * N_BB + bb) * CC
                gather_transpose_out(c0, bb, use)
                return 0

            lax.fori_loop(0, N_BB, bb_body, 0)
            return 0

        lax.fori_loop(0, N_CCHUNK, chunk_body, 0)
        lastc0 = (N_CCHUNK - 1) * CC
        lastb0 = b_w + (N_BB - 1) * BB
        for tb in range(2):
            pltpu.make_async_copy(
                t_v.at[tb], out_hbm.at[lastc0 + 2 + tb, :, pl.ds(lastb0, BB)],
                osem[tb]).wait()

    return mre_kernel


_MRE = _make_kernel()


def kernel(x, W0, W1, W2):
    tflat = jnp.concatenate(
        [W0.reshape(-1), W1.reshape(-1), W2.reshape(-1)])
    out = _MRE(x.T, tflat)
    return out.transpose(2, 0, 1)


# speedup vs baseline: 28.5675x; 3.8326x over previous
import functools

import jax
import jax.numpy as jnp
import numpy as np
from jax import lax
from jax.experimental import pallas as pl
from jax.experimental.pallas import tpu as pltpu
from jax.experimental.pallas import tpu_sc as plsc

N_CH = 100
RESOLUTIONS = (16, 64, 256)
DIM = 16
BATCH = 16384
NRES = 3
NDIM = NRES * DIM
ROWW = 18

NUM_CORES = 2
NUM_SUBCORES = 16
NW = NUM_CORES * NUM_SUBCORES
B_PER_W = BATCH // NW
BB = 128
N_BB = B_PER_W // BB
CC = 4
N_CCHUNK = N_CH // CC
LANES = 16
G_PER_BB = BB // LANES

TFLAT = (0, N_CH * (RESOLUTIONS[0] + 1) * ROWW,
         N_CH * ((RESOLUTIONS[0] + 1) + (RESOLUTIONS[1] + 1)) * ROWW)
SLICE_LEN = tuple(CC * (r + 1) * ROWW for r in RESOLUTIONS)
LBASE = (0, SLICE_LEN[0], SLICE_LEN[0] + SLICE_LEN[1])
WTAB_LEN = sum(SLICE_LEN)
IDX_LEN = CC * NRES * BB


def _make_kernel():
    mesh = plsc.VectorSubcoreMesh(core_axis_name="c", subcore_axis_name="s")

    @functools.partial(
        pl.kernel,
        out_type=jax.ShapeDtypeStruct((N_CH, NDIM, BATCH), jnp.float32),
        mesh=mesh,
        compiler_params=pltpu.CompilerParams(
            use_tc_tiling_on_sc=True, needs_layout_passes=False),
        scratch_types=[
            pltpu.VMEM((N_CH, B_PER_W), jnp.float32),
            pltpu.VMEM((2 * WTAB_LEN,), jnp.float32),
            pltpu.VMEM((IDX_LEN,), jnp.int32),
            pltpu.VMEM((2, NDIM, BB), jnp.float32),
            pltpu.SemaphoreType.DMA,
            pltpu.SemaphoreType.DMA,
            pltpu.SemaphoreType.DMA,
        ],
    )
    def mre_kernel(xt_hbm, tflat_hbm, out_hbm, x_v, w_v, idx_v, t_v,
                   o0, o1, wsem):
        osem = (o0, o1)
        wid = lax.axis_index("s") * NUM_CORES + lax.axis_index("c")
        b_w = wid * B_PER_W
        iot = lax.iota(jnp.int32, LANES)

        pltpu.sync_copy(xt_hbm.at[:, pl.ds(b_w, B_PER_W)], x_v)

        def start_load_tables(c0, woff):
            for ri, res in enumerate(RESOLUTIONS):
                rl = (res + 1) * ROWW
                pltpu.async_copy(
                    tflat_hbm.at[pl.ds(TFLAT[ri] + c0 * rl, CC * rl)],
                    w_v.at[pl.ds(woff + LBASE[ri], CC * rl)], wsem)

        def wait_load_tables(c0, woff):
            for ri, res in enumerate(RESOLUTIONS):
                rl = (res + 1) * ROWW
                pltpu.make_async_copy(
                    tflat_hbm.at[pl.ds(TFLAT[ri] + c0 * rl, CC * rl)],
                    w_v.at[pl.ds(woff + LBASE[ri], CC * rl)], wsem).wait()

        def compute_idx(c0, bb, woff):
            def idx_body(i, _):
                cc = i // G_PER_BB
                g = lax.rem(i, G_PER_BB)
                xv = x_v[c0 + cc, pl.ds(bb * BB + g * LANES, LANES)]
                for ri, res in enumerate(RESOLUTIONS):
                    scale = np.float32(res - 1)
                    step = np.float32(1.0 / (res - 1))
                    rl = (res + 1) * ROWW
                    t = xv * scale
                    c0i = t.astype(jnp.int32)
                    base = c0i - 1
                    acc = jnp.maximum(base, 0)
                    for j in range(4):
                        k = base + j
                        bk = k.astype(jnp.float32) * step
                        valid = (k >= 0) & (k <= res - 1)
                        acc = acc + jnp.where(valid & (bk < xv),
                                              jnp.int32(1), jnp.int32(0))
                    flat = acc * ROWW + (LBASE[ri] + cc * rl) + woff
                    pos = (cc * NRES + ri) * BB + g * LANES
                    idx_v[pl.ds(pos, LANES)] = flat
                return 0

            lax.fori_loop(0, CC * G_PER_BB, idx_body, 0)

        def gather_transpose_out(c0, bb, use):
            b0 = b_w + bb * BB
            for cc in range(CC):
                tb = cc % 2

                @pl.when(use + cc >= 2)
                def _(tb=tb, cc=cc):
                    pltpu.make_async_copy(
                        t_v.at[tb],
                        out_hbm.at[c0 + cc, :, pl.ds(b0, BB)], osem[tb]).wait()

                def t_body(i, _, cc=cc, tb=tb):
                    ri = i // G_PER_BB
                    g = lax.rem(i, G_PER_BB)
                    ridx = idx_v[pl.ds((cc * NRES + ri) * BB + g * LANES,
                                       LANES)]
                    vs = [plsc.load_gather(w_v, [ridx + dd])
                          for dd in range(DIM)]
                    for dd in range(DIM):
                        t_v[tb, ri * DIM + dd, pl.ds(g * LANES, LANES)] = vs[dd]
                    return 0

                lax.fori_loop(0, NRES * G_PER_BB, t_body, 0)
                pltpu.async_copy(
                    t_v.at[tb], out_hbm.at[c0 + cc, :, pl.ds(b0, BB)],
                    osem[tb])

        def chunk_body(ci, _):
            c0 = ci * CC
            woff = lax.rem(ci, 2) * WTAB_LEN
            wait_load_tables(c0, woff)

            @pl.when(ci < N_CCHUNK - 1)
            def _():
                start_load_tables((ci + 1) * CC, WTAB_LEN - woff)

            def bb_body(bb, _):
                compute_idx(c0, bb, woff)
                use = (ci * N_BB + bb) * CC
                gather_transpose_out(c0, bb, use)
                return 0

            lax.fori_loop(0, N_BB, bb_body, 0)
            return 0

        start_load_tables(0, 0)
        lax.fori_loop(0, N_CCHUNK, chunk_body, 0)
        lastc0 = (N_CCHUNK - 1) * CC
        lastb0 = b_w + (N_BB - 1) * BB
        for tb in range(2):
            pltpu.make_async_copy(
                t_v.at[tb], out_hbm.at[lastc0 + 2 + tb, :, pl.ds(lastb0, BB)],
                osem[tb]).wait()

    return mre_kernel


_MRE = _make_kernel()


def kernel(x, W0, W1, W2):
    pad = ((0, 0), (0, ROWW - DIM))
    tflat = jnp.concatenate(
        [jnp.pad(W0, pad).reshape(-1), jnp.pad(W1, pad).reshape(-1),
         jnp.pad(W2, pad).reshape(-1)])
    out = _MRE(x.T, tflat)
    return out.transpose(2, 0, 1)


# speedup vs baseline: 29.3416x; 1.0271x over previous
import functools

import jax
import jax.numpy as jnp
import numpy as np
from jax import lax
from jax.experimental import pallas as pl
from jax.experimental.pallas import tpu as pltpu
from jax.experimental.pallas import tpu_sc as plsc

N_CH = 100
RESOLUTIONS = (16, 64, 256)
DIM = 16
BATCH = 16384
NRES = 3
NDIM = NRES * DIM
ROWW = 18

NUM_CORES = 2
NUM_SUBCORES = 16
NW = NUM_CORES * NUM_SUBCORES
B_PER_W = BATCH // NW
BB = 128
N_BB = B_PER_W // BB
CC = 4
N_CCHUNK = N_CH // CC
LANES = 16
G_PER_BB = BB // LANES

TFLAT = (0, N_CH * (RESOLUTIONS[0] + 1) * ROWW,
         N_CH * ((RESOLUTIONS[0] + 1) + (RESOLUTIONS[1] + 1)) * ROWW)
SLICE_LEN = tuple(CC * (r + 1) * ROWW for r in RESOLUTIONS)
LBASE = (0, SLICE_LEN[0], SLICE_LEN[0] + SLICE_LEN[1])
WTAB_LEN = sum(SLICE_LEN)
IDX_LEN = CC * NRES * BB


def _make_kernel():
    mesh = plsc.VectorSubcoreMesh(core_axis_name="c", subcore_axis_name="s")

    @functools.partial(
        pl.kernel,
        out_type=jax.ShapeDtypeStruct((N_CH, NDIM, BATCH), jnp.float32),
        mesh=mesh,
        compiler_params=pltpu.CompilerParams(
            use_tc_tiling_on_sc=True, needs_layout_passes=False),
        scratch_types=[
            pltpu.VMEM((N_CH, B_PER_W), jnp.float32),
            pltpu.VMEM((2 * WTAB_LEN,), jnp.float32),
            pltpu.VMEM((IDX_LEN,), jnp.int32),
            pltpu.VMEM((4, NDIM, BB), jnp.float32),
            pltpu.SemaphoreType.DMA,
            pltpu.SemaphoreType.DMA,
            pltpu.SemaphoreType.DMA,
            pltpu.SemaphoreType.DMA,
            pltpu.SemaphoreType.DMA,
        ],
    )
    def mre_kernel(xt_hbm, tflat_hbm, out_hbm, x_v, w_v, idx_v, t_v,
                   o0, o1, o2, o3, wsem):
        osem = (o0, o1, o2, o3)
        wid = lax.axis_index("s") * NUM_CORES + lax.axis_index("c")
        b_w = wid * B_PER_W
        iot = lax.iota(jnp.int32, LANES)

        pltpu.sync_copy(xt_hbm.at[:, pl.ds(b_w, B_PER_W)], x_v)

        def start_load_tables(c0, woff):
            for ri, res in enumerate(RESOLUTIONS):
                rl = (res + 1) * ROWW
                pltpu.async_copy(
                    tflat_hbm.at[pl.ds(TFLAT[ri] + c0 * rl, CC * rl)],
                    w_v.at[pl.ds(woff + LBASE[ri], CC * rl)], wsem)

        def wait_load_tables(c0, woff):
            for ri, res in enumerate(RESOLUTIONS):
                rl = (res + 1) * ROWW
                pltpu.make_async_copy(
                    tflat_hbm.at[pl.ds(TFLAT[ri] + c0 * rl, CC * rl)],
                    w_v.at[pl.ds(woff + LBASE[ri], CC * rl)], wsem).wait()

        def compute_idx(c0, bb, woff):
            def idx_body(i, _):
                cc = i // G_PER_BB
                g = lax.rem(i, G_PER_BB)
                xv = x_v[c0 + cc, pl.ds(bb * BB + g * LANES, LANES)]
                for ri, res in enumerate(RESOLUTIONS):
                    scale = np.float32(res - 1)
                    step = np.float32(1.0 / (res - 1))
                    rl = (res + 1) * ROWW
                    t = xv * scale
                    c0i = t.astype(jnp.int32)
                    base = c0i - 1
                    acc = jnp.maximum(base, 0)
                    cf = c0i.astype(jnp.float32)
                    ok0 = base >= 0
                    for j in range(4):
                        bk = (cf + np.float32(j - 1)) * step
                        cond = bk < xv
                        if j == 0:
                            cond = cond & ok0
                        acc = acc + jnp.where(cond, jnp.int32(1), jnp.int32(0))
                    flat = acc * ROWW + (LBASE[ri] + cc * rl) + woff
                    pos = (cc * NRES + ri) * BB + g * LANES
                    idx_v[pl.ds(pos, LANES)] = flat
                return 0

            lax.fori_loop(0, CC * G_PER_BB, idx_body, 0)

        def gather_transpose_out(c0, bb, use):
            b0 = b_w + bb * BB
            for cc in range(CC):
                tb = cc % 4

                @pl.when(use + cc >= 4)
                def _(tb=tb, cc=cc):
                    pltpu.make_async_copy(
                        t_v.at[tb],
                        out_hbm.at[c0 + cc, :, pl.ds(b0, BB)], osem[tb]).wait()

                def t_body(i, _, cc=cc, tb=tb):
                    ri = i // G_PER_BB
                    g = lax.rem(i, G_PER_BB)
                    ridx = idx_v[pl.ds((cc * NRES + ri) * BB + g * LANES,
                                       LANES)]
                    vs = [plsc.load_gather(w_v, [ridx + dd])
                          for dd in range(DIM)]
                    for dd in range(DIM):
                        t_v[tb, ri * DIM + dd, pl.ds(g * LANES, LANES)] = vs[dd]
                    return 0

                lax.fori_loop(0, NRES * G_PER_BB, t_body, 0)
                pltpu.async_copy(
                    t_v.at[tb], out_hbm.at[c0 + cc, :, pl.ds(b0, BB)],
                    osem[tb])

        def chunk_body(ci, _):
            c0 = ci * CC
            woff = lax.rem(ci, 2) * WTAB_LEN
            wait_load_tables(c0, woff)

            @pl.when(ci < N_CCHUNK - 1)
            def _():
                start_load_tables((ci + 1) * CC, WTAB_LEN - woff)

            def bb_body(bb, _):
                compute_idx(c0, bb, woff)
                use = (ci * N_BB + bb) * CC
                gather_transpose_out(c0, bb, use)
                return 0

            lax.fori_loop(0, N_BB, bb_body, 0)
            return 0

        start_load_tables(0, 0)
        lax.fori_loop(0, N_CCHUNK, chunk_body, 0)
        lastc0 = (N_CCHUNK - 1) * CC
        lastb0 = b_w + (N_BB - 1) * BB
        for tb in range(4):
            pltpu.make_async_copy(
                t_v.at[tb], out_hbm.at[lastc0 + tb, :, pl.ds(lastb0, BB)],
                osem[tb]).wait()

    return mre_kernel


_MRE = _make_kernel()


def kernel(x, W0, W1, W2):
    pad = ((0, 0), (0, ROWW - DIM))
    tflat = jnp.concatenate(
        [jnp.pad(W0, pad).reshape(-1), jnp.pad(W1, pad).reshape(-1),
         jnp.pad(W2, pad).reshape(-1)])
    out = _MRE(x.T, tflat)
    return out.transpose(2, 0, 1)


# speedup vs baseline: 33.9027x; 1.1554x over previous
import functools

import jax
import jax.numpy as jnp
import numpy as np
from jax import lax
from jax.experimental import pallas as pl
from jax.experimental.pallas import tpu as pltpu
from jax.experimental.pallas import tpu_sc as plsc

N_CH = 100
RESOLUTIONS = (16, 64, 256)
DIM = 16
BATCH = 16384
NRES = 3
NDIM = NRES * DIM
ROWW = 18

NUM_CORES = 2
NUM_SUBCORES = 16
NW = NUM_CORES * NUM_SUBCORES
B_PER_W = BATCH // NW
BB = 128
N_BB = B_PER_W // BB
CC = 4
N_CCHUNK = N_CH // CC
LANES = 16
G_PER_BB = BB // LANES

TFLAT = (0, N_CH * (RESOLUTIONS[0] + 1) * ROWW,
         N_CH * ((RESOLUTIONS[0] + 1) + (RESOLUTIONS[1] + 1)) * ROWW)
SLICE_LEN = tuple(CC * (r + 1) * ROWW for r in RESOLUTIONS)
LBASE = (0, SLICE_LEN[0], SLICE_LEN[0] + SLICE_LEN[1])
WTAB_LEN = sum(SLICE_LEN)
IDX_LEN = CC * NRES * BB


def _make_kernel():
    mesh = plsc.VectorSubcoreMesh(core_axis_name="c", subcore_axis_name="s")

    @functools.partial(
        pl.kernel,
        out_type=jax.ShapeDtypeStruct((N_CH, NDIM, BATCH), jnp.float32),
        mesh=mesh,
        compiler_params=pltpu.CompilerParams(
            use_tc_tiling_on_sc=True, needs_layout_passes=False),
        scratch_types=[
            pltpu.VMEM((N_CH, B_PER_W), jnp.float32),
            pltpu.VMEM((2 * WTAB_LEN,), jnp.float32),
            pltpu.VMEM((IDX_LEN,), jnp.int32),
            pltpu.VMEM((4, NDIM, BB), jnp.float32),
            pltpu.SemaphoreType.DMA,
            pltpu.SemaphoreType.DMA,
            pltpu.SemaphoreType.DMA,
            pltpu.SemaphoreType.DMA,
            pltpu.SemaphoreType.DMA,
        ],
    )
    def mre_kernel(xt_hbm, tflat_hbm, out_hbm, x_v, w_v, idx_v, t_v,
                   o0, o1, o2, o3, wsem):
        osem = (o0, o1, o2, o3)
        wid = lax.axis_index("s") * NUM_CORES + lax.axis_index("c")
        b_w = wid * B_PER_W
        iot = lax.iota(jnp.int32, LANES)

        pltpu.sync_copy(xt_hbm.at[:, pl.ds(b_w, B_PER_W)], x_v)

        def start_load_tables(c0, woff):
            for ri, res in enumerate(RESOLUTIONS):
                rl = (res + 1) * ROWW
                pltpu.async_copy(
                    tflat_hbm.at[pl.ds(TFLAT[ri] + c0 * rl, CC * rl)],
                    w_v.at[pl.ds(woff + LBASE[ri], CC * rl)], wsem)

        def wait_load_tables(c0, woff):
            for ri, res in enumerate(RESOLUTIONS):
                rl = (res + 1) * ROWW
                pltpu.make_async_copy(
                    tflat_hbm.at[pl.ds(TFLAT[ri] + c0 * rl, CC * rl)],
                    w_v.at[pl.ds(woff + LBASE[ri], CC * rl)], wsem).wait()

        def compute_idx(c0, bb, woff):
            def idx_body(g, cc):
                xv = x_v[c0 + cc, pl.ds(bb * BB + g * LANES, LANES)]
                for ri, res in enumerate(RESOLUTIONS):
                    scale = np.float32(res - 1)
                    step = np.float32(1.0 / (res - 1))
                    rl = (res + 1) * ROWW
                    t = xv * scale
                    c0i = t.astype(jnp.int32)
                    base = c0i - 1
                    acc = jnp.maximum(base, 0)
                    cf = c0i.astype(jnp.float32)
                    ok0 = base >= 0
                    for j in range(4):
                        bk = (cf + np.float32(j - 1)) * step
                        cond = bk < xv
                        if j == 0:
                            cond = cond & ok0
                        acc = acc + jnp.where(cond, jnp.int32(1), jnp.int32(0))
                    flat = acc * ROWW + (LBASE[ri] + cc * rl) + woff
                    pos = (cc * NRES + ri) * BB + g * LANES
                    idx_v[pl.ds(pos, LANES)] = flat

            for cc in range(CC):
                plsc.parallel_loop(0, G_PER_BB, unroll=2)(
                    functools.partial(idx_body, cc=cc))

        def gather_transpose_out(c0, bb, use):
            b0 = b_w + bb * BB
            for cc in range(CC):
                tb = cc % 4

                @pl.when(use + cc >= 4)
                def _(tb=tb, cc=cc):
                    pltpu.make_async_copy(
                        t_v.at[tb],
                        out_hbm.at[c0 + cc, :, pl.ds(b0, BB)], osem[tb]).wait()

                def t_body(g, cc, tb, ri):
                    ridx = idx_v[pl.ds((cc * NRES + ri) * BB + g * LANES,
                                       LANES)]
                    vs = [plsc.load_gather(w_v, [ridx + dd])
                          for dd in range(DIM)]
                    for dd in range(DIM):
                        t_v[tb, ri * DIM + dd, pl.ds(g * LANES, LANES)] = vs[dd]

                for ri in range(NRES):
                    plsc.parallel_loop(0, G_PER_BB, unroll=2)(
                        functools.partial(t_body, cc=cc, tb=tb, ri=ri))
                pltpu.async_copy(
                    t_v.at[tb], out_hbm.at[c0 + cc, :, pl.ds(b0, BB)],
                    osem[tb])

        def chunk_body(ci, _):
            c0 = ci * CC
            woff = lax.rem(ci, 2) * WTAB_LEN
            wait_load_tables(c0, woff)

            @pl.when(ci < N_CCHUNK - 1)
            def _():
                start_load_tables((ci + 1) * CC, WTAB_LEN - woff)

            def bb_body(bb, _):
                compute_idx(c0, bb, woff)
                use = (ci * N_BB + bb) * CC
                gather_transpose_out(c0, bb, use)
                return 0

            lax.fori_loop(0, N_BB, bb_body, 0)
            return 0

        start_load_tables(0, 0)
        lax.fori_loop(0, N_CCHUNK, chunk_body, 0)
        lastc0 = (N_CCHUNK - 1) * CC
        lastb0 = b_w + (N_BB - 1) * BB
        for tb in range(4):
            pltpu.make_async_copy(
                t_v.at[tb], out_hbm.at[lastc0 + tb, :, pl.ds(lastb0, BB)],
                osem[tb]).wait()

    return mre_kernel


_MRE = _make_kernel()


def kernel(x, W0, W1, W2):
    pad = ((0, 0), (0, ROWW - DIM))
    tflat = jnp.concatenate(
        [jnp.pad(W0, pad).reshape(-1), jnp.pad(W1, pad).reshape(-1),
         jnp.pad(W2, pad).reshape(-1)])
    out = _MRE(x.T, tflat)
    return out.transpose(2, 0, 1)
